# baseline (speedup 1.0000x reference)
import jax
import jax.numpy as jnp
from jax import lax
from jax.experimental import pallas as pl
from jax.experimental.pallas import tpu as pltpu
from jax.experimental.pallas import tpu_sc as plsc

NSTEP = 20
NFIELD = 26
NEMB = 64
P = NSTEP * NFIELD
BSZ = 1024
LANES = 16
HALF = P // 2
NSPH = NSTEP // 2

NC = 2
NS = 16
NW = NC * NS
B_PER_W = BSZ // NW
NUNIT = 2 * B_PER_W
NSLOT = 4


def _body(x_hbm, feat_hbm, glob_hbm, out_hbm, glob_v, rows_v, idx_v, gsem, wsem):
    wid = lax.axis_index("s") * NC + lax.axis_index("c")
    b0 = wid * B_PER_W

    pltpu.sync_copy(glob_hbm, glob_v)

    def stage_idx_and_gather(slot, u_batch, k):
        half = k % 2
        pltpu.sync_copy(
            x_hbm.at[u_batch, pl.ds(half * NSPH, NSPH)], idx_v.at[slot]
        )
        for c in range(NSPH):
            pltpu.async_copy(
                feat_hbm.at[idx_v.at[slot, c]],
                rows_v.at[slot, pl.ds(c * NFIELD, NFIELD)],
                gsem.at[slot],
            )

    def wait_gathers(slot):
        for c in range(NSPH):
            pltpu.make_async_copy(
                feat_hbm.at[idx_v.at[slot, c]],
                rows_v.at[slot, pl.ds(c * NFIELD, NFIELD)],
                gsem.at[slot],
            ).wait()

    def wait_write(slot, u_batch, k):
        pltpu.make_async_copy(
            rows_v.at[slot],
            out_hbm.at[u_batch, pl.ds((k % 2) * HALF, HALF)],
            wsem.at[slot],
        ).wait()

    stage_idx_and_gather(0, b0, 0)
    stage_idx_and_gather(1, b0, 1)

    def per_group(j, carry):
        b = b0 + 2 * j

        for k in range(NSLOT):
            u_batch = b + k // 2
            half = k % 2
            wait_gathers(k)

            base = half * HALF

            def add_row(p, c2, _k=k, _base=base):
                for s in range(NEMB // LANES):
                    sl = pl.ds(s * LANES, LANES)
                    rows_v[_k, p, sl] = rows_v[_k, p, sl] + glob_v[_base + p, sl]
                return c2

            lax.fori_loop(0, HALF, add_row, 0)

            pltpu.async_copy(
                rows_v.at[k],
                out_hbm.at[u_batch, pl.ds(base, HALF)],
                wsem.at[k],
            )

            ks = (k + 2) % NSLOT
            if k < 2:

                @pl.when(j >= 1)
                def _(_ks=ks, _k=k):
                    wait_write(_ks, b - 1, _k)

                stage_idx_and_gather(ks, b + 1, k)
            else:
                wait_write(ks, b, k)

                @pl.when(j < B_PER_W // 2 - 1)
                def _(_ks=ks, _k=k):
                    stage_idx_and_gather(_ks, b + 1 + _k // 2, _k)
        return carry

    lax.fori_loop(0, B_PER_W // 2, per_group, 0)

    blast = b0 + B_PER_W - 1
    wait_write(2, blast, 2)
    wait_write(3, blast, 3)


@jax.jit
def kernel(x, feat_table, global_table):
    mesh = plsc.VectorSubcoreMesh(core_axis_name="c", subcore_axis_name="s")
    run = pl.kernel(
        _body,
        out_type=jax.ShapeDtypeStruct((BSZ, P, NEMB), jnp.float32),
        mesh=mesh,
        compiler_params=pltpu.CompilerParams(use_tc_tiling_on_sc=False),
        scratch_types=[
            pltpu.VMEM((P, NEMB), jnp.float32),
            pltpu.VMEM((NSLOT, HALF, NEMB), jnp.float32),
            pltpu.VMEM((NSLOT, NSPH, NFIELD), jnp.int32),
            pltpu.SemaphoreType.DMA((NSLOT,)),
            pltpu.SemaphoreType.DMA((NSLOT,)),
        ],
    )
    return run(x, feat_table, global_table)


# speedup vs baseline: 1.0279x; 1.0279x over previous
import jax
import jax.numpy as jnp
from jax import lax
from jax.experimental import pallas as pl
from jax.experimental.pallas import tpu as pltpu
from jax.experimental.pallas import tpu_sc as plsc

NSTEP = 20
NFIELD = 26
NEMB = 64
P = NSTEP * NFIELD
BSZ = 1024
LANES = 16
HALF = P // 2
NSPH = NSTEP // 2
GCHUNKS = ((0, 64), (64, 64), (128, 64), (192, 64), (256, 4))

NC = 2
NS = 16
NW = NC * NS
B_PER_W = BSZ // NW
NUNIT = 2 * B_PER_W
NSLOT = 4


def _body(xt_hbm, feat_hbm, glob_hbm, out_hbm, glob_v, rows_v, nat_v, idx2_v,
          gsem, wsem):
    wid = lax.axis_index("s") * NC + lax.axis_index("c")
    b0 = wid * B_PER_W

    pltpu.sync_copy(glob_hbm, glob_v)

    lanes = lax.broadcasted_iota(jnp.int32, (LANES,), 0)
    for h in range(2):
        pltpu.sync_copy(
            xt_hbm.at[:, pl.ds(h * NSPH, NSPH), pl.ds(b0, B_PER_W)], nat_v
        )

        def repack_f(f, c2, _h=h):
            for sp in range(NSPH):
                for b16 in range(B_PER_W // LANES):
                    v = nat_v[f, sp, pl.ds(b16 * LANES, LANES)]
                    row = 2 * (lanes + b16 * LANES) + _h
                    col = jnp.full((LANES,), sp * NFIELD, jnp.int32) + f
                    plsc.store_scatter(idx2_v, [row, col], v)
            return c2

        lax.fori_loop(0, NFIELD, repack_f, 0)

    def fire_gathers(slot, u):
        for off, ln in GCHUNKS:
            pltpu.async_copy(
                feat_hbm.at[idx2_v.at[u, pl.ds(off, ln)]],
                rows_v.at[slot, pl.ds(off, ln)],
                gsem.at[slot],
            )

    def wait_gathers(slot, u):
        for off, ln in GCHUNKS:
            pltpu.make_async_copy(
                feat_hbm.at[idx2_v.at[u, pl.ds(off, ln)]],
                rows_v.at[slot, pl.ds(off, ln)],
                gsem.at[slot],
            ).wait()

    def wait_write(slot, u_batch, k):
        pltpu.make_async_copy(
            rows_v.at[slot],
            out_hbm.at[u_batch, pl.ds((k % 2) * HALF, HALF)],
            wsem.at[slot],
        ).wait()

    fire_gathers(0, 0)
    fire_gathers(1, 1)

    def per_group(j, carry):
        b = b0 + 2 * j

        for k in range(NSLOT):
            u_batch = b + k // 2
            half = k % 2
            wait_gathers(k, 4 * j + k)

            base = half * HALF

            def add_row(p, c2, _k=k, _base=base):
                for s in range(NEMB // LANES):
                    sl = pl.ds(s * LANES, LANES)
                    rows_v[_k, p, sl] = rows_v[_k, p, sl] + glob_v[_base + p, sl]
                return c2

            lax.fori_loop(0, HALF, add_row, 0)

            pltpu.async_copy(
                rows_v.at[k],
                out_hbm.at[u_batch, pl.ds(base, HALF)],
                wsem.at[k],
            )

            ks = (k + 2) % NSLOT
            if k < 2:

                @pl.when(j >= 1)
                def _(_ks=ks, _k=k):
                    wait_write(_ks, b - 1, _k)

                fire_gathers(ks, 4 * j + k + 2)
            else:
                wait_write(ks, b, k)

                @pl.when(j < B_PER_W // 2 - 1)
                def _(_ks=ks, _k=k, _j=j):
                    fire_gathers(_ks, 4 * _j + _k + 2)
        return carry

    lax.fori_loop(0, B_PER_W // 2, per_group, 0)

    blast = b0 + B_PER_W - 1
    wait_write(2, blast, 2)
    wait_write(3, blast, 3)


@jax.jit
def kernel(x, feat_table, global_table):
    xt = jnp.transpose(x, (2, 1, 0))
    mesh = plsc.VectorSubcoreMesh(core_axis_name="c", subcore_axis_name="s")
    run = pl.kernel(
        _body,
        out_type=jax.ShapeDtypeStruct((BSZ, P, NEMB), jnp.float32),
        mesh=mesh,
        compiler_params=pltpu.CompilerParams(
            use_tc_tiling_on_sc=False, needs_layout_passes=False
        ),
        scratch_types=[
            pltpu.VMEM((P, NEMB), jnp.float32),
            pltpu.VMEM((NSLOT, HALF, NEMB), jnp.float32),
            pltpu.VMEM((NFIELD, NSPH, B_PER_W), jnp.int32),
            pltpu.VMEM((NUNIT, HALF), jnp.int32),
            pltpu.SemaphoreType.DMA((NSLOT,)),
            pltpu.SemaphoreType.DMA((NSLOT,)),
        ],
    )
    return run(xt, feat_table, global_table)
